# baseline (speedup 1.0000x reference)
import functools

import jax
import jax.numpy as jnp
from jax.experimental import pallas as pl
from jax.experimental.pallas import tpu as pltpu
from jax.experimental.pallas import tpu_sc as plsc

_WINDOW = 800


def kernel(x, table):
    batch, hist = x.shape
    n = batch * hist
    embed = table.shape[1]
    idx = x.T.reshape(1, n).astype(jnp.int32)
    mesh = plsc.VectorSubcoreMesh(core_axis_name="c", subcore_axis_name="s")

    sub = _WINDOW // 2

    @functools.partial(
        pl.kernel,
        out_type=jax.ShapeDtypeStruct((n, embed), table.dtype),
        mesh=mesh,
        scratch_types=[pltpu.SemaphoreType.DMA((2,))],
        compiler_params=pltpu.CompilerParams(use_tc_tiling_on_sc=False),
    )
    def gather_kernel(table_hbm, i_hbm, o_hbm, sems):
        def body(i_vmem, o_vmem):
            for j in range(2):
                pltpu.async_copy(
                    table_hbm.at[i_vmem.at[0, pl.ds(j * sub, sub)]],
                    o_vmem.at[pl.ds(j * sub, sub)],
                    sems.at[j],
                )
            for j in range(2):
                pltpu.make_async_copy(
                    table_hbm.at[i_vmem.at[0, pl.ds(j * sub, sub)]],
                    o_vmem.at[pl.ds(j * sub, sub)],
                    sems.at[j],
                ).wait()

        pltpu.emit_pipeline(
            body,
            grid=(n // _WINDOW,),
            in_specs=[
                pl.BlockSpec((1, _WINDOW), index_map=lambda i: (0, i))
            ],
            out_specs=[
                pl.BlockSpec((_WINDOW, embed), index_map=lambda i: (i, 0))
            ],
            core_axis_name=("c", "s"),
            dimension_semantics=(pltpu.PARALLEL,),
        )(i_hbm, o_hbm)

    out_cm = gather_kernel(table, idx)
    return out_cm.reshape(hist, batch, embed).transpose(1, 0, 2)


# speedup vs baseline: 1.0010x; 1.0010x over previous
import functools

import jax
import jax.numpy as jnp
from jax.experimental import pallas as pl
from jax.experimental.pallas import tpu as pltpu
from jax.experimental.pallas import tpu_sc as plsc

_WINDOW = 512


def kernel(x, table):
    batch, hist = x.shape
    n = batch * hist
    embed = table.shape[1]
    idx = x.T.reshape(1, n).astype(jnp.int32)
    mesh = plsc.VectorSubcoreMesh(core_axis_name="c", subcore_axis_name="s")

    sub = _WINDOW // 2

    @functools.partial(
        pl.kernel,
        out_type=jax.ShapeDtypeStruct((n, embed), table.dtype),
        mesh=mesh,
        scratch_types=[pltpu.SemaphoreType.DMA((2,))],
        compiler_params=pltpu.CompilerParams(use_tc_tiling_on_sc=False),
    )
    def gather_kernel(table_hbm, i_hbm, o_hbm, sems):
        def body(i_vmem, o_vmem):
            for j in range(2):
                pltpu.async_copy(
                    table_hbm.at[i_vmem.at[0, pl.ds(j * sub, sub)]],
                    o_vmem.at[pl.ds(j * sub, sub)],
                    sems.at[j],
                )
            for j in range(2):
                pltpu.make_async_copy(
                    table_hbm.at[i_vmem.at[0, pl.ds(j * sub, sub)]],
                    o_vmem.at[pl.ds(j * sub, sub)],
                    sems.at[j],
                ).wait()

        pltpu.emit_pipeline(
            body,
            grid=(n // _WINDOW,),
            in_specs=[
                pl.BlockSpec((1, _WINDOW), index_map=lambda i: (0, i))
            ],
            out_specs=[
                pl.BlockSpec((_WINDOW, embed), index_map=lambda i: (i, 0))
            ],
            core_axis_name=("c", "s"),
            dimension_semantics=(pltpu.PARALLEL,),
        )(i_hbm, o_hbm)

    out_cm = gather_kernel(table, idx)
    return out_cm.reshape(hist, batch, embed).transpose(1, 0, 2)
